# Initial kernel scaffold; baseline (speedup 1.0000x reference)
import jax
import jax.numpy as jnp
from jax.experimental import pallas as pl


def kernel(x, y, post_graphs):
    raise NotImplementedError("write your pallas kernel here")


# speedup vs baseline: 2.6297x; 2.6297x over previous
import numpy as np

import jax
import jax.numpy as jnp
from jax.experimental import pallas as pl

_P = 3
_SIZES = np.array([2, 2, 2])
_OFFS = np.concatenate([[0], np.cumsum(_SIZES)[:-1]]).astype(np.int32)
_NPOST = int(_SIZES.sum())
_NCOMBO = int(np.prod(_SIZES))
_NPAT = 2 ** _P
_NK = _NPAT * _NCOMBO
_C = 64
_CC = _C * _C
_TB = 128

_COMBO_SEL = np.stack(
    np.meshgrid(*[np.arange(int(s)) for s in _SIZES], indexing="ij"), -1
).reshape(-1, _P).astype(np.int32)

_K_GRAPHS = []
for _p in range(_NPAT):
    for _c in range(_NCOMBO):
        _K_GRAPHS.append(
            [int(_OFFS[j] + _COMBO_SEL[_c, j]) for j in range(_P) if (_p >> j) & 1]
        )


def _tables_kernel(pg_ref, gflat_ref, rflat_ref, pen_ref):
    pg = pg_ref[...]
    row = jax.lax.broadcasted_iota(jnp.int32, (_C, _C), 0)
    col = jax.lax.broadcasted_iota(jnp.int32, (_C, _C), 1)
    eye = (row == col).astype(jnp.float32)

    gs = []
    for sel in _K_GRAPHS:
        if sel:
            gk = pg[sel[0]]
            for g in sel[1:]:
                gk = jnp.maximum(gk, pg[g])
        else:
            gk = jnp.zeros((_C, _C), jnp.float32)
        gs.append(gk)
    G = jnp.stack(gs, axis=0)

    M = jnp.minimum(G + eye[None], 1.0)
    dn = (((2,), (1,)), ((0,), (0,)))
    for _ in range(6):
        M = (jax.lax.dot_general(M, M, dn, preferred_element_type=jnp.float32)
             > 0).astype(jnp.float32)
    S = jax.lax.dot_general(M, M, dn, preferred_element_type=jnp.float32)
    tr = jnp.sum(jnp.sum(S * eye[None], axis=2), axis=1, keepdims=True)
    pen = jnp.where(tr > _C + 0.5, jnp.float32(1e9), 0.0)
    ones_row = jnp.ones((1, _NK), jnp.float32)
    pen_row = jnp.dot(ones_row, pen * jnp.broadcast_to(eye, (_NK, _NK)),
                      preferred_element_type=jnp.float32)

    gflat_ref[...] = G.reshape(_NK, _CC)
    rflat_ref[...] = M.reshape(_NK, _CC)
    pen_ref[...] = jnp.broadcast_to(pen_row, (8, _NK))


def _repair_kernel(xs_ref, y_ref, gflat_ref, rflat_ref, pen_ref,
                   y2_ref, bot_ref):
    y = y_ref[...]
    xs = xs_ref[...]
    gflat = gflat_ref[...]
    rflat = rflat_ref[...]
    pen_row = pen_ref[...][0:1, :]

    patt = jnp.zeros((_TB, 1), jnp.int32)
    for p in range(_P):
        sp = jnp.sum(xs[:, 8 * p:8 * (p + 1)], axis=1, keepdims=True)
        patt = patt + (sp > 0).astype(jnp.int32) * (1 << p)

    V = jax.nn.relu(y[:, None, :] - y[:, :, None]).reshape(_TB, _CC)
    nt = (((1,), (1,)), ((), ()))
    viol = jax.lax.dot_general(V, gflat, nt,
                               preferred_element_type=jnp.float32)
    score = viol + pen_row

    kio = jax.lax.broadcasted_iota(jnp.int32, (_TB, _NK), 1)
    in_patt = (kio // _NCOMBO) == patt
    big = jnp.float32(3e38)
    score_m = jnp.where(in_patt, score, big)
    mn = jnp.min(score_m, axis=1, keepdims=True)
    kb = jnp.min(jnp.where(score_m == mn, kio, _NK), axis=1, keepdims=True)
    onehot = (kio == kb).astype(jnp.float32)

    Vpos = (V > 0).astype(jnp.float32)
    cnt = jax.lax.dot_general(Vpos, gflat, nt,
                              preferred_element_type=jnp.float32)
    cnt_sel = jnp.sum(jnp.where(kio == kb, cnt, 0.0), axis=1, keepdims=True)
    sat = cnt_sel == 0

    rb = jnp.dot(onehot, rflat,
                 preferred_element_type=jnp.float32).reshape(_TB, _C, _C)
    yfx = jnp.max(jnp.where(rb > 0, y[:, None, :], jnp.float32(-1e30)), axis=2)
    y2 = jnp.where(sat, y, yfx)

    V2 = jax.nn.relu(y2[:, None, :] - y2[:, :, None]).reshape(_TB, _CC)
    mg = jax.lax.dot_general(V2, gflat, nt,
                             preferred_element_type=jnp.float32)
    margin = jnp.sum(jnp.where(kio == kb, mg, 0.0), axis=1, keepdims=True)
    ymax = jnp.max(y2, axis=1, keepdims=True)
    bot = jnp.where(margin > 0, ymax + margin, -jnp.inf)

    y2_ref[...] = y2
    bot_ref[...] = jnp.broadcast_to(bot, (_TB, _C))


def kernel(x, y, post_graphs):
    b = y.shape[0]
    pg = post_graphs.astype(jnp.float32)
    gflat, rflat, pen = pl.pallas_call(
        _tables_kernel,
        out_shape=(
            jax.ShapeDtypeStruct((_NK, _CC), jnp.float32),
            jax.ShapeDtypeStruct((_NK, _CC), jnp.float32),
            jax.ShapeDtypeStruct((8, _NK), jnp.float32),
        ),
    )(pg)

    xs = x[:, : 8 * _P]
    grid = b // _TB
    y2, bot = pl.pallas_call(
        _repair_kernel,
        grid=(grid,),
        in_specs=[
            pl.BlockSpec((_TB, 8 * _P), lambda i: (i, 0)),
            pl.BlockSpec((_TB, _C), lambda i: (i, 0)),
            pl.BlockSpec((_NK, _CC), lambda i: (0, 0)),
            pl.BlockSpec((_NK, _CC), lambda i: (0, 0)),
            pl.BlockSpec((8, _NK), lambda i: (0, 0)),
        ],
        out_specs=(
            pl.BlockSpec((_TB, _C), lambda i: (i, 0)),
            pl.BlockSpec((_TB, _C), lambda i: (i, 0)),
        ),
        out_shape=(
            jax.ShapeDtypeStruct((b, _C), jnp.float32),
            jax.ShapeDtypeStruct((b, _C), jnp.float32),
        ),
    )(xs, y, gflat, rflat, pen)
    return jnp.concatenate([y2, bot[:, :1]], axis=1)
